# baseline (speedup 1.0000x reference)
import functools

import jax
import jax.numpy as jnp
from jax import lax
from jax.experimental import pallas as pl
from jax.experimental.pallas import tpu as pltpu
from jax.experimental.pallas import tpu_sc as plsc

_N = 10000
_NP = 10112
_E = 320000
_NC = 2
_NS = 16
_L = 16
_B = 112
_NBATCH = 90
_EPT = _NBATCH * _B
_EP = _NC * _NS * _EPT
_RPT = _NP // _NS
_DEGW = 16
_NSLOT = 3
_NB0 = 108
_NB1 = 72

_mesh = lambda: plsc.VectorSubcoreMesh(core_axis_name="c", subcore_axis_name="s")


@functools.partial(
    pl.kernel,
    out_type=jax.ShapeDtypeStruct((_NC, _NP, _DEGW), jnp.float32),
    mesh=_mesh(),
    scratch_types=[
        pltpu.VMEM((_B, _DEGW), jnp.float32),
        pltpu.VMEM((_NBATCH, _B), jnp.int32),
        pltpu.VMEM((_NBATCH, _B), jnp.int32),
        pltpu.VMEM_SHARED((_NP, _DEGW), jnp.float32),
    ],
    compiler_params=pltpu.CompilerParams(use_tc_tiling_on_sc=False),
)
def _sc_degree(row_hbm, col_hbm, w_hbm, out_hbm, wrow_v, cidx_all, w_all, dacc):
    c = lax.axis_index("c")
    s = lax.axis_index("s")
    tile = c * _NS + s

    pltpu.sync_copy(col_hbm.at[pl.ds(tile * _NBATCH, _NBATCH)], cidx_all)
    pltpu.sync_copy(w_hbm.at[pl.ds(tile * _NBATCH, _NBATCH)], w_all)

    def zb(i, carry):
        wrow_v[i, :] = jnp.zeros((_DEGW,), jnp.float32)
        return carry

    lax.fori_loop(0, _B, zb, 0)
    for k in range(5):
        pltpu.sync_copy(wrow_v, dacc.at[pl.ds(s * _RPT + k * _B, _B)])
    pltpu.sync_copy(wrow_v.at[pl.ds(0, _RPT - 5 * _B)],
                    dacc.at[pl.ds(s * _RPT + 5 * _B, _RPT - 5 * _B)])
    plsc.subcore_barrier()

    def batch(bi, carry):
        def group(g, carry2):
            wv = lax.bitcast_convert_type(
                w_all[bi, pl.ds(g * _L, _L)], jnp.float32)
            for jj in range(_L):
                wrow_v[g * _L + jj, :] = jnp.full((_DEGW,), wv[jj], jnp.float32)
            return carry2

        lax.fori_loop(0, _B // _L, group, 0)
        pltpu.sync_copy(wrow_v, dacc.at[cidx_all.at[bi]], add=True)
        return carry

    lax.fori_loop(0, _NBATCH, batch, 0)
    plsc.subcore_barrier()
    pltpu.sync_copy(dacc.at[pl.ds(s * _RPT, _RPT)],
                    out_hbm.at[c, pl.ds(s * _RPT, _RPT), :])


def _make_sc_agg(F):
    nch = F // _L

    @functools.partial(
        pl.kernel,
        out_type=jax.ShapeDtypeStruct((_NC, _NP, F), jnp.float32),
        mesh=_mesh(),
        scratch_types=[
            [pltpu.VMEM((_B, F), jnp.float32) for _ in range(_NSLOT)],
            [pltpu.VMEM((3, _B), jnp.int32) for _ in range(2 * _NSLOT)],
            pltpu.VMEM_SHARED((_NP, F), jnp.float32),
            [pltpu.SemaphoreType.DMA for _ in range(_NSLOT)],
            [pltpu.SemaphoreType.DMA for _ in range(_NSLOT)],
            [pltpu.SemaphoreType.DMA for _ in range(2 * _NSLOT)],
        ],
        compiler_params=pltpu.CompilerParams(use_tc_tiling_on_sc=False),
    )
    def agg(h_hbm, row_hbm, col_hbm, w_hbm, out_hbm,
            bufs, idxs, acc, gsems, ssems, isems):
        c = lax.axis_index("c")
        s = lax.axis_index("s")
        nb = jnp.where(c == 0, _NB0, _NB1)
        base = c * (_NS * _NB0) + s * nb

        def zb(i, carry):
            for k in range(nch):
                bufs[0][i, pl.ds(k * _L, _L)] = jnp.zeros((_L,), jnp.float32)
            return carry

        lax.fori_loop(0, _B, zb, 0)
        for k in range(5):
            pltpu.sync_copy(bufs[0], acc.at[pl.ds(s * _RPT + k * _B, _B)])
        pltpu.sync_copy(bufs[0].at[pl.ds(0, _RPT - 5 * _B)],
                        acc.at[pl.ds(s * _RPT + 5 * _B, _RPT - 5 * _B)])
        plsc.subcore_barrier()

        def start_idx(q, bi):
            pltpu.async_copy(row_hbm.at[base + bi], idxs[q].at[0], isems[q])
            pltpu.async_copy(col_hbm.at[base + bi], idxs[q].at[1], isems[q])
            pltpu.async_copy(w_hbm.at[base + bi], idxs[q].at[2], isems[q])

        def wait_idx(q, bi):
            pltpu.make_async_copy(row_hbm.at[base + bi], idxs[q].at[0],
                                  isems[q]).wait()
            pltpu.make_async_copy(col_hbm.at[base + bi], idxs[q].at[1],
                                  isems[q]).wait()
            pltpu.make_async_copy(w_hbm.at[base + bi], idxs[q].at[2],
                                  isems[q]).wait()

        def start_gather(p, q):
            pltpu.async_copy(h_hbm.at[idxs[q].at[0]], bufs[p], gsems[p])

        def wait_gather(p, q):
            pltpu.make_async_copy(h_hbm.at[idxs[q].at[0]], bufs[p],
                                  gsems[p]).wait()

        def start_scatter(p, q):
            pltpu.async_copy(bufs[p], acc.at[idxs[q].at[1]],
                             ssems[p], add=True)

        def wait_scatter(p, q):
            pltpu.make_async_copy(bufs[p], acc.at[idxs[q].at[1]],
                                  ssems[p]).wait()

        def scale(p, q):
            def group(g, carry2):
                wv = lax.bitcast_convert_type(
                    idxs[q][2, pl.ds(g * _L, _L)], jnp.float32)
                for jj in range(_L):
                    sj = wv[jj]
                    j = g * _L + jj
                    for k in range(nch):
                        sl = pl.ds(k * _L, _L)
                        bufs[p][j, sl] = bufs[p][j, sl] * sj
                return carry2

            lax.fori_loop(0, _B // _L, group, 0)

        for q in range(4):
            start_idx(q, q)
        for p in range(2):
            wait_idx(p, p)
            start_gather(p, p)

        nq = 2 * _NSLOT

        def outer(gi, carry):
            for pp in range(nq):
                b = gi * nq + pp
                p = pp % _NSLOT
                q = pp
                q2 = (pp + 2) % nq
                q4 = (pp + 4) % nq
                qm1 = (pp + nq - 1) % nq
                wait_gather(p, q)
                scale(p, q)
                start_scatter(p, q)
                p2 = (p + 2) % _NSLOT

                @pl.when(b + 4 < nb)
                def _():
                    start_idx(q4, b + 4)

                @pl.when(b + 2 < nb)
                def _():
                    @pl.when(b >= 1)
                    def _():
                        wait_scatter(p2, qm1)
                    wait_idx(q2, b + 2)
                    start_gather(p2, q2)
            return carry

        lax.fori_loop(0, nb // nq, outer, 0)
        wait_scatter(0, 3)
        wait_scatter(1, 4)
        wait_scatter(2, 5)
        plsc.subcore_barrier()
        pltpu.sync_copy(acc.at[pl.ds(s * _RPT, _RPT)],
                        out_hbm.at[c, pl.ds(s * _RPT, _RPT), :])

    return agg


_sc_agg128 = _make_sc_agg(128)
_sc_agg64 = _make_sc_agg(64)


_RB = 1264


def _mm1_body(x_ref, w_ref, d_ref, h_ref, dis_ref):
    deg = d_ref[0][:, 0:1] + d_ref[1][:, 0:1] + 1.0
    dis = jnp.broadcast_to(lax.rsqrt(deg), (x_ref.shape[0], 128))
    dis_ref[...] = dis
    h = jnp.dot(x_ref[...], w_ref[...], preferred_element_type=jnp.float32)
    h_ref[...] = h * dis


def _mm1(x, W, dparts):
    return pl.pallas_call(
        _mm1_body,
        out_shape=[jax.ShapeDtypeStruct((_NP, 128), jnp.float32),
                   jax.ShapeDtypeStruct((_NP, 128), jnp.float32)],
        grid=(_NP // _RB,),
        in_specs=[pl.BlockSpec((_RB, 128), lambda i: (i, 0)),
                  pl.BlockSpec((128, 128), lambda i: (0, 0)),
                  pl.BlockSpec((2, _RB, _DEGW), lambda i: (0, i, 0))],
        out_specs=[pl.BlockSpec((_RB, 128), lambda i: (i, 0)),
                   pl.BlockSpec((_RB, 128), lambda i: (i, 0))],
    )(x, W, dparts)


def _comb_body(p_ref, h_ref, dis_ref, b_ref, w_ref, out_ref):
    x = (dis_ref[...] * (p_ref[0] + p_ref[1] + h_ref[...]) + b_ref[...])
    x = jnp.maximum(x, 0.0)
    h2 = jnp.dot(x, w_ref[...], preferred_element_type=jnp.float32)
    out_ref[...] = h2 * dis_ref[...][:, :out_ref.shape[1]]


def _comb(p, h, dis, b, W, f_out):
    return pl.pallas_call(
        _comb_body,
        out_shape=jax.ShapeDtypeStruct((_NP, f_out), jnp.float32),
        grid=(_NP // _RB,),
        in_specs=[pl.BlockSpec((2, _RB, 128), lambda i: (0, i, 0)),
                  pl.BlockSpec((_RB, 128), lambda i: (i, 0)),
                  pl.BlockSpec((_RB, 128), lambda i: (i, 0)),
                  pl.BlockSpec((1, 128), lambda i: (0, 0)),
                  pl.BlockSpec((128, f_out), lambda i: (0, 0))],
        out_specs=pl.BlockSpec((_RB, f_out), lambda i: (i, 0)),
    )(p, h, dis, b, W)


def _final_body(p_ref, h_ref, dis_ref, b_ref, out_ref):
    out_ref[...] = (dis_ref[...][:, :out_ref.shape[1]]
                    * (p_ref[0] + p_ref[1] + h_ref[...]) + b_ref[...])


def _final(p, h, dis, b):
    rb = 1000
    return pl.pallas_call(
        _final_body,
        out_shape=jax.ShapeDtypeStruct((_N, 64), jnp.float32),
        grid=(_N // rb,),
        in_specs=[pl.BlockSpec((2, rb, 64), lambda i: (0, i, 0)),
                  pl.BlockSpec((rb, 64), lambda i: (i, 0)),
                  pl.BlockSpec((rb, 128), lambda i: (i, 0)),
                  pl.BlockSpec((1, 64), lambda i: (0, 0))],
        out_specs=pl.BlockSpec((rb, 64), lambda i: (i, 0)),
    )(p, h, dis, b)


def kernel(features, edge_index, edge_weight, W1, b1, W2, b2, W3, b3):
    pad = _EP - _E
    nbt = _NC * _NS * _NBATCH
    feats = jnp.concatenate(
        [features, jnp.zeros((_NP - _N, features.shape[1]), features.dtype)])
    rowp = jnp.concatenate(
        [edge_index[0], jnp.zeros((pad,), edge_index.dtype)]).reshape(nbt, _B)
    colp = jnp.concatenate(
        [edge_index[1], jnp.zeros((pad,), edge_index.dtype)]).reshape(nbt, _B)
    wbits = jax.lax.bitcast_convert_type(
        jnp.concatenate([edge_weight, jnp.zeros((pad,), edge_weight.dtype)]),
        jnp.int32).reshape(nbt, _B)

    dparts = _sc_degree(rowp, colp, wbits)
    h1, dis = _mm1(feats, W1, dparts)
    p1 = _sc_agg128(h1, rowp, colp, wbits)
    h2 = _comb(p1, h1, dis, b1.reshape(1, 128), W2, 128)
    p2 = _sc_agg128(h2, rowp, colp, wbits)
    h3 = _comb(p2, h2, dis, b2.reshape(1, 128), W3, 64)
    p3 = _sc_agg64(h3, rowp, colp, wbits)
    return _final(p3, h3, dis, b3.reshape(1, 64))


# speedup vs baseline: 1.1025x; 1.1025x over previous
import functools

import jax
import jax.numpy as jnp
from jax import lax
from jax.experimental import pallas as pl
from jax.experimental.pallas import tpu as pltpu
from jax.experimental.pallas import tpu_sc as plsc

_N = 10000
_NP = 10112
_E = 320000
_NC = 2
_NS = 16
_L = 16
_B = 112
_NBATCH = 90
_EPT = _NBATCH * _B
_EP = _NC * _NS * _EPT
_RPT = _NP // _NS
_DEGW = 16
_NSLOT = 3
_NB0 = 102
_NB1 = 78

_mesh = lambda: plsc.VectorSubcoreMesh(core_axis_name="c", subcore_axis_name="s")


@functools.partial(
    pl.kernel,
    out_type=jax.ShapeDtypeStruct((_NC, _NP, _DEGW), jnp.float32),
    mesh=_mesh(),
    scratch_types=[
        pltpu.VMEM((_B, _DEGW), jnp.float32),
        pltpu.VMEM((_NBATCH, _B), jnp.int32),
        pltpu.VMEM((_NBATCH, _B), jnp.int32),
        pltpu.VMEM_SHARED((_NP, _DEGW), jnp.float32),
    ],
    compiler_params=pltpu.CompilerParams(use_tc_tiling_on_sc=False),
)
def _sc_degree(row_hbm, col_hbm, w_hbm, out_hbm, wrow_v, cidx_all, w_all, dacc):
    c = lax.axis_index("c")
    s = lax.axis_index("s")
    tile = c * _NS + s

    pltpu.sync_copy(col_hbm.at[pl.ds(tile * _NBATCH, _NBATCH)], cidx_all)
    pltpu.sync_copy(w_hbm.at[pl.ds(tile * _NBATCH, _NBATCH)], w_all)

    def zb(i, carry):
        wrow_v[i, :] = jnp.zeros((_DEGW,), jnp.float32)
        return carry

    lax.fori_loop(0, _B, zb, 0)
    for k in range(5):
        pltpu.sync_copy(wrow_v, dacc.at[pl.ds(s * _RPT + k * _B, _B)])
    pltpu.sync_copy(wrow_v.at[pl.ds(0, _RPT - 5 * _B)],
                    dacc.at[pl.ds(s * _RPT + 5 * _B, _RPT - 5 * _B)])
    plsc.subcore_barrier()

    def batch(bi, carry):
        def group(g, carry2):
            wv = lax.bitcast_convert_type(
                w_all[bi, pl.ds(g * _L, _L)], jnp.float32)
            for jj in range(_L):
                wrow_v[g * _L + jj, :] = jnp.full((_DEGW,), wv[jj], jnp.float32)
            return carry2

        lax.fori_loop(0, _B // _L, group, 0)
        pltpu.sync_copy(wrow_v, dacc.at[cidx_all.at[bi]], add=True)
        return carry

    lax.fori_loop(0, _NBATCH, batch, 0)
    plsc.subcore_barrier()
    pltpu.sync_copy(dacc.at[pl.ds(s * _RPT, _RPT)],
                    out_hbm.at[c, pl.ds(s * _RPT, _RPT), :])


def _make_sc_agg(F):
    nch = F // _L

    @functools.partial(
        pl.kernel,
        out_type=jax.ShapeDtypeStruct((_NC, _NP, F), jnp.float32),
        mesh=_mesh(),
        scratch_types=[
            [pltpu.VMEM((_B, F), jnp.float32) for _ in range(_NSLOT)],
            [pltpu.VMEM((3, _B), jnp.int32) for _ in range(2 * _NSLOT)],
            pltpu.VMEM_SHARED((_NP, F), jnp.float32),
            [pltpu.SemaphoreType.DMA for _ in range(_NSLOT)],
            [pltpu.SemaphoreType.DMA for _ in range(_NSLOT)],
            [pltpu.SemaphoreType.DMA for _ in range(2 * _NSLOT)],
        ],
        compiler_params=pltpu.CompilerParams(use_tc_tiling_on_sc=False),
    )
    def agg(h_hbm, row_hbm, col_hbm, w_hbm, out_hbm,
            bufs, idxs, acc, gsems, ssems, isems):
        c = lax.axis_index("c")
        s = lax.axis_index("s")
        nb = jnp.where(c == 0, _NB0, _NB1)
        base = c * (_NS * _NB0) + s * nb

        def zb(i, carry):
            for k in range(nch):
                bufs[0][i, pl.ds(k * _L, _L)] = jnp.zeros((_L,), jnp.float32)
            return carry

        lax.fori_loop(0, _B, zb, 0)
        for k in range(5):
            pltpu.sync_copy(bufs[0], acc.at[pl.ds(s * _RPT + k * _B, _B)])
        pltpu.sync_copy(bufs[0].at[pl.ds(0, _RPT - 5 * _B)],
                        acc.at[pl.ds(s * _RPT + 5 * _B, _RPT - 5 * _B)])
        plsc.subcore_barrier()

        def start_idx(q, bi):
            pltpu.async_copy(row_hbm.at[base + bi], idxs[q].at[0], isems[q])
            pltpu.async_copy(col_hbm.at[base + bi], idxs[q].at[1], isems[q])
            pltpu.async_copy(w_hbm.at[base + bi], idxs[q].at[2], isems[q])

        def wait_idx(q, bi):
            pltpu.make_async_copy(row_hbm.at[base + bi], idxs[q].at[0],
                                  isems[q]).wait()
            pltpu.make_async_copy(col_hbm.at[base + bi], idxs[q].at[1],
                                  isems[q]).wait()
            pltpu.make_async_copy(w_hbm.at[base + bi], idxs[q].at[2],
                                  isems[q]).wait()

        def start_gather(p, q):
            pltpu.async_copy(h_hbm.at[idxs[q].at[0]], bufs[p], gsems[p])

        def wait_gather(p, q):
            pltpu.make_async_copy(h_hbm.at[idxs[q].at[0]], bufs[p],
                                  gsems[p]).wait()

        def start_scatter(p, q):
            pltpu.async_copy(bufs[p], acc.at[idxs[q].at[1]],
                             ssems[p], add=True)

        def wait_scatter(p, q):
            pltpu.make_async_copy(bufs[p], acc.at[idxs[q].at[1]],
                                  ssems[p]).wait()

        def scale(p, q):
            @plsc.parallel_loop(0, _B // _L)
            def group(g):
                wv = lax.bitcast_convert_type(
                    idxs[q][2, pl.ds(g * _L, _L)], jnp.float32)
                for jj in range(_L):
                    sj = wv[jj]
                    j = g * _L + jj
                    for k in range(nch):
                        sl = pl.ds(k * _L, _L)
                        bufs[p][j, sl] = bufs[p][j, sl] * sj

        for q in range(4):
            start_idx(q, q)
        for p in range(2):
            wait_idx(p, p)
            start_gather(p, p)

        nq = 2 * _NSLOT

        def outer(gi, carry):
            for pp in range(nq):
                b = gi * nq + pp
                p = pp % _NSLOT
                q = pp
                q2 = (pp + 2) % nq
                q4 = (pp + 4) % nq
                qm1 = (pp + nq - 1) % nq
                wait_gather(p, q)
                scale(p, q)
                start_scatter(p, q)
                p2 = (p + 2) % _NSLOT

                @pl.when(b + 4 < nb)
                def _():
                    start_idx(q4, b + 4)

                @pl.when(b + 2 < nb)
                def _():
                    @pl.when(b >= 1)
                    def _():
                        wait_scatter(p2, qm1)
                    wait_idx(q2, b + 2)
                    start_gather(p2, q2)
            return carry

        lax.fori_loop(0, nb // nq, outer, 0)
        wait_scatter(0, 3)
        wait_scatter(1, 4)
        wait_scatter(2, 5)
        plsc.subcore_barrier()
        pltpu.sync_copy(acc.at[pl.ds(s * _RPT, _RPT)],
                        out_hbm.at[c, pl.ds(s * _RPT, _RPT), :])

    return agg


_sc_agg128 = _make_sc_agg(128)
_sc_agg64 = _make_sc_agg(64)


_RB = 1264


def _mm1_body(x_ref, w_ref, d_ref, h_ref, dis_ref):
    deg = d_ref[0][:, 0:1] + d_ref[1][:, 0:1] + 1.0
    dis = jnp.broadcast_to(lax.rsqrt(deg), (x_ref.shape[0], 128))
    dis_ref[...] = dis
    h = jnp.dot(x_ref[...], w_ref[...], preferred_element_type=jnp.float32)
    h_ref[...] = h * dis


def _mm1(x, W, dparts):
    return pl.pallas_call(
        _mm1_body,
        out_shape=[jax.ShapeDtypeStruct((_NP, 128), jnp.float32),
                   jax.ShapeDtypeStruct((_NP, 128), jnp.float32)],
        grid=(_NP // _RB,),
        in_specs=[pl.BlockSpec((_RB, 128), lambda i: (i, 0)),
                  pl.BlockSpec((128, 128), lambda i: (0, 0)),
                  pl.BlockSpec((2, _RB, _DEGW), lambda i: (0, i, 0))],
        out_specs=[pl.BlockSpec((_RB, 128), lambda i: (i, 0)),
                   pl.BlockSpec((_RB, 128), lambda i: (i, 0))],
    )(x, W, dparts)


def _comb_body(p_ref, h_ref, dis_ref, b_ref, w_ref, out_ref):
    x = (dis_ref[...] * (p_ref[0] + p_ref[1] + h_ref[...]) + b_ref[...])
    x = jnp.maximum(x, 0.0)
    h2 = jnp.dot(x, w_ref[...], preferred_element_type=jnp.float32)
    out_ref[...] = h2 * dis_ref[...][:, :out_ref.shape[1]]


def _comb(p, h, dis, b, W, f_out):
    return pl.pallas_call(
        _comb_body,
        out_shape=jax.ShapeDtypeStruct((_NP, f_out), jnp.float32),
        grid=(_NP // _RB,),
        in_specs=[pl.BlockSpec((2, _RB, 128), lambda i: (0, i, 0)),
                  pl.BlockSpec((_RB, 128), lambda i: (i, 0)),
                  pl.BlockSpec((_RB, 128), lambda i: (i, 0)),
                  pl.BlockSpec((1, 128), lambda i: (0, 0)),
                  pl.BlockSpec((128, f_out), lambda i: (0, 0))],
        out_specs=pl.BlockSpec((_RB, f_out), lambda i: (i, 0)),
    )(p, h, dis, b, W)


def _final_body(p_ref, h_ref, dis_ref, b_ref, out_ref):
    out_ref[...] = (dis_ref[...][:, :out_ref.shape[1]]
                    * (p_ref[0] + p_ref[1] + h_ref[...]) + b_ref[...])


def _final(p, h, dis, b):
    rb = 1000
    return pl.pallas_call(
        _final_body,
        out_shape=jax.ShapeDtypeStruct((_N, 64), jnp.float32),
        grid=(_N // rb,),
        in_specs=[pl.BlockSpec((2, rb, 64), lambda i: (0, i, 0)),
                  pl.BlockSpec((rb, 64), lambda i: (i, 0)),
                  pl.BlockSpec((rb, 128), lambda i: (i, 0)),
                  pl.BlockSpec((1, 64), lambda i: (0, 0))],
        out_specs=pl.BlockSpec((rb, 64), lambda i: (i, 0)),
    )(p, h, dis, b)


def kernel(features, edge_index, edge_weight, W1, b1, W2, b2, W3, b3):
    pad = _EP - _E
    nbt = _NC * _NS * _NBATCH
    feats = jnp.concatenate(
        [features, jnp.zeros((_NP - _N, features.shape[1]), features.dtype)])
    rowp = jnp.concatenate(
        [edge_index[0], jnp.zeros((pad,), edge_index.dtype)]).reshape(nbt, _B)
    colp = jnp.concatenate(
        [edge_index[1], jnp.zeros((pad,), edge_index.dtype)]).reshape(nbt, _B)
    wbits = jax.lax.bitcast_convert_type(
        jnp.concatenate([edge_weight, jnp.zeros((pad,), edge_weight.dtype)]),
        jnp.int32).reshape(nbt, _B)

    dparts = _sc_degree(rowp, colp, wbits)
    h1, dis = _mm1(feats, W1, dparts)
    p1 = _sc_agg128(h1, rowp, colp, wbits)
    h2 = _comb(p1, h1, dis, b1.reshape(1, 128), W2, 128)
    p2 = _sc_agg128(h2, rowp, colp, wbits)
    h3 = _comb(p2, h2, dis, b2.reshape(1, 128), W3, 64)
    p3 = _sc_agg64(h3, rowp, colp, wbits)
    return _final(p3, h3, dis, b3.reshape(1, 64))


# speedup vs baseline: 1.1529x; 1.0457x over previous
import functools

import jax
import jax.numpy as jnp
from jax import lax
from jax.experimental import pallas as pl
from jax.experimental.pallas import tpu as pltpu
from jax.experimental.pallas import tpu_sc as plsc

_N = 10000
_NP = 10112
_E = 320000
_NC = 2
_NS = 16
_L = 16
_B = 112
_NBATCH = 90
_EPT = _NBATCH * _B
_EP = _NC * _NS * _EPT
_RPT = _NP // _NS
_DEGW = 16
_NSLOT = 3
_NB0 = 114
_NB1 = 66

_mesh = lambda: plsc.VectorSubcoreMesh(core_axis_name="c", subcore_axis_name="s")


@functools.partial(
    pl.kernel,
    out_type=jax.ShapeDtypeStruct((_NC, _NP, _DEGW), jnp.float32),
    mesh=_mesh(),
    scratch_types=[
        pltpu.VMEM((_B, _DEGW), jnp.float32),
        pltpu.VMEM((_NBATCH, _B), jnp.int32),
        pltpu.VMEM((_NBATCH, _B), jnp.int32),
        pltpu.VMEM_SHARED((_NP, _DEGW), jnp.float32),
    ],
    compiler_params=pltpu.CompilerParams(use_tc_tiling_on_sc=False),
)
def _sc_degree(row_hbm, col_hbm, w_hbm, out_hbm, wrow_v, cidx_all, w_all, dacc):
    c = lax.axis_index("c")
    s = lax.axis_index("s")
    tile = c * _NS + s

    pltpu.sync_copy(col_hbm.at[pl.ds(tile * _NBATCH, _NBATCH)], cidx_all)
    pltpu.sync_copy(w_hbm.at[pl.ds(tile * _NBATCH, _NBATCH)], w_all)

    def zb(i, carry):
        wrow_v[i, :] = jnp.zeros((_DEGW,), jnp.float32)
        return carry

    lax.fori_loop(0, _B, zb, 0)
    for k in range(5):
        pltpu.sync_copy(wrow_v, dacc.at[pl.ds(s * _RPT + k * _B, _B)])
    pltpu.sync_copy(wrow_v.at[pl.ds(0, _RPT - 5 * _B)],
                    dacc.at[pl.ds(s * _RPT + 5 * _B, _RPT - 5 * _B)])
    plsc.subcore_barrier()

    def batch(bi, carry):
        def group(g, carry2):
            wv = lax.bitcast_convert_type(
                w_all[bi, pl.ds(g * _L, _L)], jnp.float32)
            for jj in range(_L):
                wrow_v[g * _L + jj, :] = jnp.full((_DEGW,), wv[jj], jnp.float32)
            return carry2

        lax.fori_loop(0, _B // _L, group, 0)
        pltpu.sync_copy(wrow_v, dacc.at[cidx_all.at[bi]], add=True)
        return carry

    lax.fori_loop(0, _NBATCH, batch, 0)
    plsc.subcore_barrier()
    pltpu.sync_copy(dacc.at[pl.ds(s * _RPT, _RPT)],
                    out_hbm.at[c, pl.ds(s * _RPT, _RPT), :])


def _make_sc_agg(F):
    nch = F // _L

    @functools.partial(
        pl.kernel,
        out_type=jax.ShapeDtypeStruct((_NC, _NP, F), jnp.float32),
        mesh=_mesh(),
        scratch_types=[
            [pltpu.VMEM((_B, F), jnp.float32) for _ in range(_NSLOT)],
            [pltpu.VMEM((3, _B), jnp.int32) for _ in range(2 * _NSLOT)],
            pltpu.VMEM_SHARED((_NP, F), jnp.float32),
            [pltpu.SemaphoreType.DMA for _ in range(_NSLOT)],
            [pltpu.SemaphoreType.DMA for _ in range(_NSLOT)],
            [pltpu.SemaphoreType.DMA for _ in range(2 * _NSLOT)],
        ],
        compiler_params=pltpu.CompilerParams(use_tc_tiling_on_sc=False),
    )
    def agg(h_hbm, row_hbm, col_hbm, w_hbm, out_hbm,
            bufs, idxs, acc, gsems, ssems, isems):
        c = lax.axis_index("c")
        s = lax.axis_index("s")
        nb = jnp.where(c == 0, _NB0, _NB1)
        base = c * (_NS * _NB0) + s * nb

        def zb(i, carry):
            for k in range(nch):
                bufs[0][i, pl.ds(k * _L, _L)] = jnp.zeros((_L,), jnp.float32)
            return carry

        lax.fori_loop(0, _B, zb, 0)
        for k in range(5):
            pltpu.sync_copy(bufs[0], acc.at[pl.ds(s * _RPT + k * _B, _B)])
        pltpu.sync_copy(bufs[0].at[pl.ds(0, _RPT - 5 * _B)],
                        acc.at[pl.ds(s * _RPT + 5 * _B, _RPT - 5 * _B)])
        plsc.subcore_barrier()

        def start_idx(q, bi):
            pltpu.async_copy(row_hbm.at[base + bi], idxs[q].at[0], isems[q])
            pltpu.async_copy(col_hbm.at[base + bi], idxs[q].at[1], isems[q])
            pltpu.async_copy(w_hbm.at[base + bi], idxs[q].at[2], isems[q])

        def wait_idx(q, bi):
            pltpu.make_async_copy(row_hbm.at[base + bi], idxs[q].at[0],
                                  isems[q]).wait()
            pltpu.make_async_copy(col_hbm.at[base + bi], idxs[q].at[1],
                                  isems[q]).wait()
            pltpu.make_async_copy(w_hbm.at[base + bi], idxs[q].at[2],
                                  isems[q]).wait()

        def start_gather(p, q):
            pltpu.async_copy(h_hbm.at[idxs[q].at[0]], bufs[p], gsems[p])

        def wait_gather(p, q):
            pltpu.make_async_copy(h_hbm.at[idxs[q].at[0]], bufs[p],
                                  gsems[p]).wait()

        def start_scatter(p, q):
            pltpu.async_copy(bufs[p], acc.at[idxs[q].at[1]],
                             ssems[p], add=True)

        def wait_scatter(p, q):
            pltpu.make_async_copy(bufs[p], acc.at[idxs[q].at[1]],
                                  ssems[p]).wait()

        def scale(p, q):
            @plsc.parallel_loop(0, _B // _L)
            def group(g):
                wv = lax.bitcast_convert_type(
                    idxs[q][2, pl.ds(g * _L, _L)], jnp.float32)
                for jj in range(_L):
                    sj = wv[jj]
                    j = g * _L + jj
                    for k in range(nch):
                        sl = pl.ds(k * _L, _L)
                        bufs[p][j, sl] = bufs[p][j, sl] * sj

        for q in range(4):
            start_idx(q, q)
        for p in range(2):
            wait_idx(p, p)
            start_gather(p, p)

        nq = 2 * _NSLOT

        def outer(gi, carry):
            for pp in range(nq):
                b = gi * nq + pp
                p = pp % _NSLOT
                q = pp
                q2 = (pp + 2) % nq
                q4 = (pp + 4) % nq
                qm1 = (pp + nq - 1) % nq
                wait_gather(p, q)
                scale(p, q)
                start_scatter(p, q)
                p2 = (p + 2) % _NSLOT

                @pl.when(b + 4 < nb)
                def _():
                    start_idx(q4, b + 4)

                @pl.when(b + 2 < nb)
                def _():
                    @pl.when(b >= 1)
                    def _():
                        wait_scatter(p2, qm1)
                    wait_idx(q2, b + 2)
                    start_gather(p2, q2)
            return carry

        lax.fori_loop(0, nb // nq, outer, 0)
        wait_scatter(0, 3)
        wait_scatter(1, 4)
        wait_scatter(2, 5)
        plsc.subcore_barrier()
        pltpu.sync_copy(acc.at[pl.ds(s * _RPT, _RPT)],
                        out_hbm.at[c, pl.ds(s * _RPT, _RPT), :])

    return agg


_sc_agg128 = _make_sc_agg(128)
_sc_agg64 = _make_sc_agg(64)


_RB = 1264


def _mm1_body(x_ref, w_ref, d_ref, h_ref, dis_ref):
    deg = d_ref[0][:, 0:1] + d_ref[1][:, 0:1] + 1.0
    dis = jnp.broadcast_to(lax.rsqrt(deg), (x_ref.shape[0], 128))
    dis_ref[...] = dis
    h = jnp.dot(x_ref[...], w_ref[...], preferred_element_type=jnp.float32)
    h_ref[...] = h * dis


def _mm1(x, W, dparts):
    return pl.pallas_call(
        _mm1_body,
        out_shape=[jax.ShapeDtypeStruct((_NP, 128), jnp.float32),
                   jax.ShapeDtypeStruct((_NP, 128), jnp.float32)],
        grid=(_NP // _RB,),
        in_specs=[pl.BlockSpec((_RB, 128), lambda i: (i, 0)),
                  pl.BlockSpec((128, 128), lambda i: (0, 0)),
                  pl.BlockSpec((2, _RB, _DEGW), lambda i: (0, i, 0))],
        out_specs=[pl.BlockSpec((_RB, 128), lambda i: (i, 0)),
                   pl.BlockSpec((_RB, 128), lambda i: (i, 0))],
    )(x, W, dparts)


def _comb_body(p_ref, h_ref, dis_ref, b_ref, w_ref, out_ref):
    x = (dis_ref[...] * (p_ref[0] + p_ref[1] + h_ref[...]) + b_ref[...])
    x = jnp.maximum(x, 0.0)
    h2 = jnp.dot(x, w_ref[...], preferred_element_type=jnp.float32)
    out_ref[...] = h2 * dis_ref[...][:, :out_ref.shape[1]]


def _comb(p, h, dis, b, W, f_out):
    return pl.pallas_call(
        _comb_body,
        out_shape=jax.ShapeDtypeStruct((_NP, f_out), jnp.float32),
        grid=(_NP // _RB,),
        in_specs=[pl.BlockSpec((2, _RB, 128), lambda i: (0, i, 0)),
                  pl.BlockSpec((_RB, 128), lambda i: (i, 0)),
                  pl.BlockSpec((_RB, 128), lambda i: (i, 0)),
                  pl.BlockSpec((1, 128), lambda i: (0, 0)),
                  pl.BlockSpec((128, f_out), lambda i: (0, 0))],
        out_specs=pl.BlockSpec((_RB, f_out), lambda i: (i, 0)),
    )(p, h, dis, b, W)


def _final_body(p_ref, h_ref, dis_ref, b_ref, out_ref):
    out_ref[...] = (dis_ref[...][:, :out_ref.shape[1]]
                    * (p_ref[0] + p_ref[1] + h_ref[...]) + b_ref[...])


def _final(p, h, dis, b):
    rb = 1000
    return pl.pallas_call(
        _final_body,
        out_shape=jax.ShapeDtypeStruct((_N, 64), jnp.float32),
        grid=(_N // rb,),
        in_specs=[pl.BlockSpec((2, rb, 64), lambda i: (0, i, 0)),
                  pl.BlockSpec((rb, 64), lambda i: (i, 0)),
                  pl.BlockSpec((rb, 128), lambda i: (i, 0)),
                  pl.BlockSpec((1, 64), lambda i: (0, 0))],
        out_specs=pl.BlockSpec((rb, 64), lambda i: (i, 0)),
    )(p, h, dis, b)


def kernel(features, edge_index, edge_weight, W1, b1, W2, b2, W3, b3):
    pad = _EP - _E
    nbt = _NC * _NS * _NBATCH
    feats = jnp.concatenate(
        [features, jnp.zeros((_NP - _N, features.shape[1]), features.dtype)])
    rowp = jnp.concatenate(
        [edge_index[0], jnp.zeros((pad,), edge_index.dtype)]).reshape(nbt, _B)
    colp = jnp.concatenate(
        [edge_index[1], jnp.zeros((pad,), edge_index.dtype)]).reshape(nbt, _B)
    wbits = jax.lax.bitcast_convert_type(
        jnp.concatenate([edge_weight, jnp.zeros((pad,), edge_weight.dtype)]),
        jnp.int32).reshape(nbt, _B)

    dparts = _sc_degree(rowp, colp, wbits)
    h1, dis = _mm1(feats, W1, dparts)
    p1 = _sc_agg128(h1, rowp, colp, wbits)
    h2 = _comb(p1, h1, dis, b1.reshape(1, 128), W2, 128)
    p2 = _sc_agg128(h2, rowp, colp, wbits)
    h3 = _comb(p2, h2, dis, b2.reshape(1, 128), W3, 64)
    p3 = _sc_agg64(h3, rowp, colp, wbits)
    return _final(p3, h3, dis, b3.reshape(1, 64))


# speedup vs baseline: 1.1760x; 1.0200x over previous
import functools

import jax
import jax.numpy as jnp
from jax import lax
from jax.experimental import pallas as pl
from jax.experimental.pallas import tpu as pltpu
from jax.experimental.pallas import tpu_sc as plsc

_N = 10000
_NP = 10112
_E = 320000
_NC = 2
_NS = 16
_L = 16
_B = 112
_NBATCH = 90
_EPT = _NBATCH * _B
_EP = _NC * _NS * _EPT
_RPT = _NP // _NS
_DEGW = 16
_NSLOT = 3
_NB0 = 120
_NB1 = 60

_mesh = lambda: plsc.VectorSubcoreMesh(core_axis_name="c", subcore_axis_name="s")


@functools.partial(
    pl.kernel,
    out_type=jax.ShapeDtypeStruct((_NC, _NP, _DEGW), jnp.float32),
    mesh=_mesh(),
    scratch_types=[
        pltpu.VMEM((_B, _DEGW), jnp.float32),
        pltpu.VMEM((_NBATCH, _B), jnp.int32),
        pltpu.VMEM((_NBATCH, _B), jnp.int32),
        pltpu.VMEM_SHARED((_NP, _DEGW), jnp.float32),
    ],
    compiler_params=pltpu.CompilerParams(use_tc_tiling_on_sc=False),
)
def _sc_degree(row_hbm, col_hbm, w_hbm, out_hbm, wrow_v, cidx_all, w_all, dacc):
    c = lax.axis_index("c")
    s = lax.axis_index("s")
    tile = c * _NS + s

    pltpu.sync_copy(col_hbm.at[pl.ds(tile * _NBATCH, _NBATCH)], cidx_all)
    pltpu.sync_copy(w_hbm.at[pl.ds(tile * _NBATCH, _NBATCH)], w_all)

    def zb(i, carry):
        wrow_v[i, :] = jnp.zeros((_DEGW,), jnp.float32)
        return carry

    lax.fori_loop(0, _B, zb, 0)
    for k in range(5):
        pltpu.sync_copy(wrow_v, dacc.at[pl.ds(s * _RPT + k * _B, _B)])
    pltpu.sync_copy(wrow_v.at[pl.ds(0, _RPT - 5 * _B)],
                    dacc.at[pl.ds(s * _RPT + 5 * _B, _RPT - 5 * _B)])
    plsc.subcore_barrier()

    def batch(bi, carry):
        def group(g, carry2):
            wv = lax.bitcast_convert_type(
                w_all[bi, pl.ds(g * _L, _L)], jnp.float32)
            for jj in range(_L):
                wrow_v[g * _L + jj, :] = jnp.full((_DEGW,), wv[jj], jnp.float32)
            return carry2

        lax.fori_loop(0, _B // _L, group, 0)
        pltpu.sync_copy(wrow_v, dacc.at[cidx_all.at[bi]], add=True)
        return carry

    lax.fori_loop(0, _NBATCH, batch, 0)
    plsc.subcore_barrier()
    pltpu.sync_copy(dacc.at[pl.ds(s * _RPT, _RPT)],
                    out_hbm.at[c, pl.ds(s * _RPT, _RPT), :])


def _make_sc_agg(F):
    nch = F // _L

    @functools.partial(
        pl.kernel,
        out_type=jax.ShapeDtypeStruct((_NC, _NP, F), jnp.float32),
        mesh=_mesh(),
        scratch_types=[
            [pltpu.VMEM((_B, F), jnp.float32) for _ in range(_NSLOT)],
            [pltpu.VMEM((3, _B), jnp.int32) for _ in range(2 * _NSLOT)],
            pltpu.VMEM_SHARED((_NP, F), jnp.float32),
            [pltpu.SemaphoreType.DMA for _ in range(_NSLOT)],
            [pltpu.SemaphoreType.DMA for _ in range(_NSLOT)],
            [pltpu.SemaphoreType.DMA for _ in range(2 * _NSLOT)],
        ],
        compiler_params=pltpu.CompilerParams(use_tc_tiling_on_sc=False),
    )
    def agg(h_hbm, row_hbm, col_hbm, w_hbm, out_hbm,
            bufs, idxs, acc, gsems, ssems, isems):
        c = lax.axis_index("c")
        s = lax.axis_index("s")
        nb = jnp.where(c == 0, _NB0, _NB1)
        base = c * (_NS * _NB0) + s * nb

        def zb(i, carry):
            for k in range(nch):
                bufs[0][i, pl.ds(k * _L, _L)] = jnp.zeros((_L,), jnp.float32)
            return carry

        lax.fori_loop(0, _B, zb, 0)
        for k in range(5):
            pltpu.sync_copy(bufs[0], acc.at[pl.ds(s * _RPT + k * _B, _B)])
        pltpu.sync_copy(bufs[0].at[pl.ds(0, _RPT - 5 * _B)],
                        acc.at[pl.ds(s * _RPT + 5 * _B, _RPT - 5 * _B)])
        plsc.subcore_barrier()

        def start_idx(q, bi):
            pltpu.async_copy(row_hbm.at[base + bi], idxs[q].at[0], isems[q])
            pltpu.async_copy(col_hbm.at[base + bi], idxs[q].at[1], isems[q])
            pltpu.async_copy(w_hbm.at[base + bi], idxs[q].at[2], isems[q])

        def wait_idx(q, bi):
            pltpu.make_async_copy(row_hbm.at[base + bi], idxs[q].at[0],
                                  isems[q]).wait()
            pltpu.make_async_copy(col_hbm.at[base + bi], idxs[q].at[1],
                                  isems[q]).wait()
            pltpu.make_async_copy(w_hbm.at[base + bi], idxs[q].at[2],
                                  isems[q]).wait()

        def start_gather(p, q):
            pltpu.async_copy(h_hbm.at[idxs[q].at[0]], bufs[p], gsems[p])

        def wait_gather(p, q):
            pltpu.make_async_copy(h_hbm.at[idxs[q].at[0]], bufs[p],
                                  gsems[p]).wait()

        def start_scatter(p, q):
            pltpu.async_copy(bufs[p], acc.at[idxs[q].at[1]],
                             ssems[p], add=True)

        def wait_scatter(p, q):
            pltpu.make_async_copy(bufs[p], acc.at[idxs[q].at[1]],
                                  ssems[p]).wait()

        def scale(p, q):
            @plsc.parallel_loop(0, _B // _L)
            def group(g):
                wv = lax.bitcast_convert_type(
                    idxs[q][2, pl.ds(g * _L, _L)], jnp.float32)
                for jj in range(_L):
                    sj = wv[jj]
                    j = g * _L + jj
                    for k in range(nch):
                        sl = pl.ds(k * _L, _L)
                        bufs[p][j, sl] = bufs[p][j, sl] * sj

        for q in range(4):
            start_idx(q, q)
        for p in range(2):
            wait_idx(p, p)
            start_gather(p, p)

        nq = 2 * _NSLOT

        def outer(gi, carry):
            for pp in range(nq):
                b = gi * nq + pp
                p = pp % _NSLOT
                q = pp
                q2 = (pp + 2) % nq
                q4 = (pp + 4) % nq
                qm1 = (pp + nq - 1) % nq
                wait_gather(p, q)
                scale(p, q)
                start_scatter(p, q)
                p2 = (p + 2) % _NSLOT

                @pl.when(b + 4 < nb)
                def _():
                    start_idx(q4, b + 4)

                @pl.when(b + 2 < nb)
                def _():
                    @pl.when(b >= 1)
                    def _():
                        wait_scatter(p2, qm1)
                    wait_idx(q2, b + 2)
                    start_gather(p2, q2)
            return carry

        lax.fori_loop(0, nb // nq, outer, 0)
        wait_scatter(0, 3)
        wait_scatter(1, 4)
        wait_scatter(2, 5)
        plsc.subcore_barrier()
        pltpu.sync_copy(acc.at[pl.ds(s * _RPT, _RPT)],
                        out_hbm.at[c, pl.ds(s * _RPT, _RPT), :])

    return agg


_sc_agg128 = _make_sc_agg(128)
_sc_agg64 = _make_sc_agg(64)


_RB = 1264


def _mm1_body(x_ref, w_ref, d_ref, h_ref, dis_ref):
    deg = d_ref[0][:, 0:1] + d_ref[1][:, 0:1] + 1.0
    dis = jnp.broadcast_to(lax.rsqrt(deg), (x_ref.shape[0], 128))
    dis_ref[...] = dis
    h = jnp.dot(x_ref[...], w_ref[...], preferred_element_type=jnp.float32)
    h_ref[...] = h * dis


def _mm1(x, W, dparts):
    return pl.pallas_call(
        _mm1_body,
        out_shape=[jax.ShapeDtypeStruct((_NP, 128), jnp.float32),
                   jax.ShapeDtypeStruct((_NP, 128), jnp.float32)],
        grid=(_NP // _RB,),
        in_specs=[pl.BlockSpec((_RB, 128), lambda i: (i, 0)),
                  pl.BlockSpec((128, 128), lambda i: (0, 0)),
                  pl.BlockSpec((2, _RB, _DEGW), lambda i: (0, i, 0))],
        out_specs=[pl.BlockSpec((_RB, 128), lambda i: (i, 0)),
                   pl.BlockSpec((_RB, 128), lambda i: (i, 0))],
    )(x, W, dparts)


def _comb_body(p_ref, h_ref, dis_ref, b_ref, w_ref, out_ref):
    x = (dis_ref[...] * (p_ref[0] + p_ref[1] + h_ref[...]) + b_ref[...])
    x = jnp.maximum(x, 0.0)
    h2 = jnp.dot(x, w_ref[...], preferred_element_type=jnp.float32)
    out_ref[...] = h2 * dis_ref[...][:, :out_ref.shape[1]]


def _comb(p, h, dis, b, W, f_out):
    return pl.pallas_call(
        _comb_body,
        out_shape=jax.ShapeDtypeStruct((_NP, f_out), jnp.float32),
        grid=(_NP // _RB,),
        in_specs=[pl.BlockSpec((2, _RB, 128), lambda i: (0, i, 0)),
                  pl.BlockSpec((_RB, 128), lambda i: (i, 0)),
                  pl.BlockSpec((_RB, 128), lambda i: (i, 0)),
                  pl.BlockSpec((1, 128), lambda i: (0, 0)),
                  pl.BlockSpec((128, f_out), lambda i: (0, 0))],
        out_specs=pl.BlockSpec((_RB, f_out), lambda i: (i, 0)),
    )(p, h, dis, b, W)


def _final_body(p_ref, h_ref, dis_ref, b_ref, out_ref):
    out_ref[...] = (dis_ref[...][:, :out_ref.shape[1]]
                    * (p_ref[0] + p_ref[1] + h_ref[...]) + b_ref[...])


def _final(p, h, dis, b):
    rb = 1000
    return pl.pallas_call(
        _final_body,
        out_shape=jax.ShapeDtypeStruct((_N, 64), jnp.float32),
        grid=(_N // rb,),
        in_specs=[pl.BlockSpec((2, rb, 64), lambda i: (0, i, 0)),
                  pl.BlockSpec((rb, 64), lambda i: (i, 0)),
                  pl.BlockSpec((rb, 128), lambda i: (i, 0)),
                  pl.BlockSpec((1, 64), lambda i: (0, 0))],
        out_specs=pl.BlockSpec((rb, 64), lambda i: (i, 0)),
    )(p, h, dis, b)


def kernel(features, edge_index, edge_weight, W1, b1, W2, b2, W3, b3):
    pad = _EP - _E
    nbt = _NC * _NS * _NBATCH
    feats = jnp.concatenate(
        [features, jnp.zeros((_NP - _N, features.shape[1]), features.dtype)])
    rowp = jnp.concatenate(
        [edge_index[0], jnp.zeros((pad,), edge_index.dtype)]).reshape(nbt, _B)
    colp = jnp.concatenate(
        [edge_index[1], jnp.zeros((pad,), edge_index.dtype)]).reshape(nbt, _B)
    wbits = jax.lax.bitcast_convert_type(
        jnp.concatenate([edge_weight, jnp.zeros((pad,), edge_weight.dtype)]),
        jnp.int32).reshape(nbt, _B)

    dparts = _sc_degree(rowp, colp, wbits)
    h1, dis = _mm1(feats, W1, dparts)
    p1 = _sc_agg128(h1, rowp, colp, wbits)
    h2 = _comb(p1, h1, dis, b1.reshape(1, 128), W2, 128)
    p2 = _sc_agg128(h2, rowp, colp, wbits)
    h3 = _comb(p2, h2, dis, b2.reshape(1, 128), W3, 64)
    p3 = _sc_agg64(h3, rowp, colp, wbits)
    return _final(p3, h3, dis, b3.reshape(1, 64))
